# baseline (speedup 1.0000x reference)
import functools

import jax
import jax.numpy as jnp
from jax import lax
from jax.experimental import pallas as pl
from jax.experimental.pallas import tpu as pltpu
from jax.experimental.pallas import tpu_sc as plsc

VOCAB = 1000000
DIM = 64
B = 16384
NEG = 20

NC = 2
NS = 16
NW = NC * NS
BPW = B // NW
NPW = B * NEG // NW
CH = 512
NCHUNK = NPW // CH

_mesh = plsc.VectorSubcoreMesh(core_axis_name="c", subcore_axis_name="s")


@functools.partial(
    pl.kernel,
    mesh=_mesh,
    out_type=[
        jax.ShapeDtypeStruct((B, DIM), jnp.float32),
        jax.ShapeDtypeStruct((B, DIM), jnp.float32),
        jax.ShapeDtypeStruct((B * NEG, DIM), jnp.float32),
    ],
    scratch_types=[
        pltpu.VMEM((BPW,), jnp.int32),
        pltpu.VMEM((NPW,), jnp.int32),
        pltpu.VMEM((CH, DIM), jnp.float32),
        pltpu.VMEM((CH, DIM), jnp.float32),
        pltpu.SemaphoreType.DMA,
        pltpu.SemaphoreType.DMA,
    ],
    compiler_params=pltpu.CompilerParams(use_tc_tiling_on_sc=False),
)
def _skipgram(iw, ow, ng, in_tab, out_tab, o1, o2, o3,
              idx_v, nidx_v, rows_a, rows_b, sem_a, sem_b):
    wid = lax.axis_index("s") * NC + lax.axis_index("c")
    base = pl.multiple_of(wid * BPW, BPW)

    pltpu.sync_copy(iw.at[pl.ds(base, BPW)], idx_v)
    pltpu.async_copy(in_tab.at[idx_v], rows_a, sem_a).wait()
    pltpu.sync_copy(rows_a, o1.at[pl.ds(base, BPW)])

    pltpu.sync_copy(ow.at[pl.ds(base, BPW)], idx_v)
    pltpu.async_copy(out_tab.at[idx_v], rows_a, sem_a).wait()
    pltpu.sync_copy(rows_a, o2.at[pl.ds(base, BPW)])

    nbase = pl.multiple_of(wid * NPW, NPW)
    pltpu.sync_copy(ng.at[pl.ds(nbase, NPW)], nidx_v)

    bufs = (rows_a, rows_b)
    sems = (sem_a, sem_b)

    pltpu.async_copy(out_tab.at[nidx_v.at[pl.ds(0, CH)]], bufs[0], sems[0])

    def step(i, _):
        cur = lax.rem(i, 2)
        nxt = 1 - cur

        @pl.when(i + 1 < NCHUNK)
        def _():
            off = pl.multiple_of((i + 1) * CH, CH)
            for b in range(2):
                @pl.when(nxt == b)
                def _():
                    pltpu.async_copy(
                        out_tab.at[nidx_v.at[pl.ds(off, CH)]], bufs[b], sems[b])

        off_i = pl.multiple_of(i * CH, CH)
        for b in range(2):
            @pl.when(cur == b)
            def _():
                pltpu.make_async_copy(out_tab.at[nidx_v.at[pl.ds(0, CH)]],
                                      bufs[b], sems[b]).wait()
                pltpu.sync_copy(bufs[b], o3.at[pl.ds(nbase + off_i, CH)])
        return 0

    lax.fori_loop(0, NCHUNK, step, 0)


def kernel(input_word, output_word, neg_samples, in_table, out_table):
    iw = input_word.reshape(B).astype(jnp.int32)
    ow = output_word.reshape(B).astype(jnp.int32)
    ng = neg_samples.reshape(B * NEG).astype(jnp.int32)
    o1, o2, o3 = _skipgram(iw, ow, ng, in_table, out_table)
    return o1, o2, o3.reshape(B, NEG, DIM)


# speedup vs baseline: 1.0020x; 1.0020x over previous
import functools

import jax
import jax.numpy as jnp
from jax import lax
from jax.experimental import pallas as pl
from jax.experimental.pallas import tpu as pltpu
from jax.experimental.pallas import tpu_sc as plsc

VOCAB = 1000000
DIM = 64
B = 16384
NEG = 20

NC = 2
NS = 16
NW = NC * NS
BPW = B // NW
NPW = B * NEG // NW
CH = 512
NJOBS = 2 + NPW // CH
NBUF = 3
DEPTH = 2

_mesh = plsc.VectorSubcoreMesh(core_axis_name="c", subcore_axis_name="s")


@functools.partial(
    pl.kernel,
    mesh=_mesh,
    out_type=[
        jax.ShapeDtypeStruct((B, DIM), jnp.float32),
        jax.ShapeDtypeStruct((B, DIM), jnp.float32),
        jax.ShapeDtypeStruct((B * NEG, DIM), jnp.float32),
    ],
    scratch_types=[
        pltpu.VMEM((2 * BPW + NPW,), jnp.int32),
        pltpu.VMEM((CH, DIM), jnp.float32),
        pltpu.VMEM((CH, DIM), jnp.float32),
        pltpu.VMEM((CH, DIM), jnp.float32),
        pltpu.SemaphoreType.DMA,
        pltpu.SemaphoreType.DMA,
        pltpu.SemaphoreType.DMA,
        pltpu.SemaphoreType.DMA,
        pltpu.SemaphoreType.DMA,
        pltpu.SemaphoreType.DMA,
        pltpu.SemaphoreType.DMA,
    ],
    compiler_params=pltpu.CompilerParams(use_tc_tiling_on_sc=False),
)
def _skipgram(iw, ow, ng, in_tab, out_tab, o1, o2, o3,
              idx_v, buf0, buf1, buf2,
              isem, g0, g1, g2, w0, w1, w2):
    wid = lax.axis_index("s") * NC + lax.axis_index("c")
    base = pl.multiple_of(wid * BPW, BPW)
    nbase = pl.multiple_of(wid * NPW, NPW)

    bufs = (buf0, buf1, buf2)
    gsems = (g0, g1, g2)
    wsems = (w0, w1, w2)

    c1 = pltpu.async_copy(iw.at[pl.ds(base, BPW)], idx_v.at[pl.ds(0, BPW)], isem)
    c2 = pltpu.async_copy(ow.at[pl.ds(base, BPW)], idx_v.at[pl.ds(BPW, BPW)], isem)
    c3 = pltpu.async_copy(ng.at[pl.ds(nbase, NPW)], idx_v.at[pl.ds(2 * BPW, NPW)], isem)
    c1.wait(); c2.wait(); c3.wait()

    jobs = [(in_tab, 0, o1, base), (out_tab, BPW, o2, base)]
    for j in range(NPW // CH):
        jobs.append((out_tab, 2 * BPW + j * CH, o3, nbase + j * CH))

    def fire_gather(j):
        tab, ioff, _, _ = jobs[j]
        return pltpu.async_copy(
            tab.at[idx_v.at[pl.ds(ioff, CH)]], bufs[j % NBUF], gsems[j % NBUF])

    def fire_writeback(j):
        _, _, out, ooff = jobs[j]
        return pltpu.async_copy(
            bufs[j % NBUF], out.at[pl.ds(ooff, CH)], wsems[j % NBUF])

    gh = [None] * NJOBS
    wh = [None] * NJOBS
    for j in range(DEPTH):
        gh[j] = fire_gather(j)
    for j in range(NJOBS):
        gh[j].wait()
        wh[j] = fire_writeback(j)
        if j + DEPTH < NJOBS:
            if j + DEPTH >= NBUF:
                wh[j + DEPTH - NBUF].wait()
            gh[j + DEPTH] = fire_gather(j + DEPTH)
    for j in range(NJOBS - DEPTH - 1, NJOBS):
        wh[j].wait()


def kernel(input_word, output_word, neg_samples, in_table, out_table):
    iw = input_word.reshape(B).astype(jnp.int32)
    ow = output_word.reshape(B).astype(jnp.int32)
    ng = neg_samples.reshape(B * NEG).astype(jnp.int32)
    o1, o2, o3 = _skipgram(iw, ow, ng, in_table, out_table)
    return o1, o2, o3.reshape(B, NEG, DIM)


# speedup vs baseline: 1.0035x; 1.0015x over previous
import functools

import jax
import jax.numpy as jnp
from jax import lax
from jax.experimental import pallas as pl
from jax.experimental.pallas import tpu as pltpu
from jax.experimental.pallas import tpu_sc as plsc

VOCAB = 1000000
DIM = 64
B = 16384
NEG = 20

NC = 2
NS = 16
NW = NC * NS
BPW = B // NW
NJOBS = 2 + NEG
NBUF = 3
DEPTH = 2

_mesh = plsc.VectorSubcoreMesh(core_axis_name="c", subcore_axis_name="s")


@functools.partial(
    pl.kernel,
    mesh=_mesh,
    out_type=[
        jax.ShapeDtypeStruct((B, DIM), jnp.float32),
        jax.ShapeDtypeStruct((B, DIM), jnp.float32),
        jax.ShapeDtypeStruct((B, NEG, DIM), jnp.float32),
    ],
    scratch_types=[
        pltpu.VMEM((2 + NEG, BPW), jnp.int32),
        pltpu.VMEM((BPW, DIM), jnp.float32),
        pltpu.VMEM((BPW, DIM), jnp.float32),
        pltpu.VMEM((BPW, DIM), jnp.float32),
        pltpu.SemaphoreType.DMA,
        pltpu.SemaphoreType.DMA,
        pltpu.SemaphoreType.DMA,
        pltpu.SemaphoreType.DMA,
        pltpu.SemaphoreType.DMA,
        pltpu.SemaphoreType.DMA,
        pltpu.SemaphoreType.DMA,
    ],
    compiler_params=pltpu.CompilerParams(use_tc_tiling_on_sc=False),
)
def _skipgram(iwt, owt, ngt, in_tab, out_tab, o1, o2, o3,
              idx_v, nb0, nb1, nb2,
              isem, g0, g1, g2, w0, w1, w2):
    wid = lax.axis_index("s") * NC + lax.axis_index("c")
    base = pl.multiple_of(wid * BPW, BPW)

    nbufs = (nb0, nb1, nb2)
    gsems = (g0, g1, g2)
    wsems = (w0, w1, w2)

    stg = [pltpu.async_copy(iwt.at[:, pl.ds(base, BPW)],
                            idx_v.at[pl.ds(0, 1), :], isem),
           pltpu.async_copy(owt.at[:, pl.ds(base, BPW)],
                            idx_v.at[pl.ds(1, 1), :], isem),
           pltpu.async_copy(ngt.at[:, pl.ds(base, BPW)],
                            idx_v.at[pl.ds(2, NEG), :], isem)]
    for h in stg:
        h.wait()

    jobs = [(idx_v.at[0], in_tab, o1.at[pl.ds(base, BPW)]),
            (idx_v.at[1], out_tab, o2.at[pl.ds(base, BPW)])]
    for j in range(NEG):
        jobs.append((idx_v.at[2 + j], out_tab, o3.at[pl.ds(base, BPW), j, :]))

    def fire_gather(j):
        offs, tab, _ = jobs[j]
        return pltpu.async_copy(
            tab.at[offs], nbufs[j % NBUF], gsems[j % NBUF])

    def fire_writeback(j):
        _, _, dst = jobs[j]
        return pltpu.async_copy(nbufs[j % NBUF], dst, wsems[j % NBUF])

    gh = [None] * NJOBS
    wh = [None] * NJOBS
    for j in range(DEPTH):
        gh[j] = fire_gather(j)
    for j in range(NJOBS):
        gh[j].wait()
        wh[j] = fire_writeback(j)
        if j + DEPTH < NJOBS:
            if j + DEPTH >= NBUF:
                wh[j + DEPTH - NBUF].wait()
            gh[j + DEPTH] = fire_gather(j + DEPTH)
    for j in range(NJOBS - DEPTH - 1, NJOBS):
        wh[j].wait()


def kernel(input_word, output_word, neg_samples, in_table, out_table):
    iwt = input_word.astype(jnp.int32).reshape(1, B)
    owt = output_word.astype(jnp.int32).reshape(1, B)
    ngt = neg_samples.astype(jnp.int32).T
    return tuple(_skipgram(iwt, owt, ngt, in_table, out_table))
